# baseline (speedup 1.0000x reference)
import functools

import jax
import jax.numpy as jnp
from jax import lax
from jax.experimental import pallas as pl
from jax.experimental.pallas import tpu as pltpu
from jax.experimental.pallas import tpu_sc as plsc

N = 10000
E = 320000
D = 128
H = 128
END = 64
OUT = 1
EPS = 1e-5

NC = 2
NS = 16
NW = NC * NS
CH = 128
EPAD = 327680
EPW = EPAD // NW
NPAD = 10240
RPT = NPAD // NS
DUMP = NPAD - 1

K = 5
GRP = K * CH
NGRP = EPW // GRP
AK = 5
ACH = 32
AGRP = AK * ACH
ANGRP = EPW // AGRP
WROWS = EPW // ACH


@functools.cache
def _sc_degree_kernel():
    mesh = plsc.VectorSubcoreMesh(core_axis_name="c", subcore_axis_name="s",
                                  num_cores=NC, num_subcores=NS)
    return pl.kernel(
        _sc_degree,
        out_type=jax.ShapeDtypeStruct((NC, NPAD, D), jnp.float32),
        mesh=mesh,
        scratch_types=[
            pltpu.VMEM((2, K, CH), jnp.int32),
            pltpu.VMEM((CH, D), jnp.float32),
            pltpu.VMEM_SHARED((NPAD, D), jnp.float32),
            pltpu.SemaphoreType.DMA,
            pltpu.SemaphoreType.DMA,
            pltpu.SemaphoreType.DMA,
        ],
    )


def _sc_degree(dst_hbm, ones_hbm, zeros_hbm, out_hbm, didx_v, ones_v, acc,
               semI, semS0, semS1):
    cid = lax.axis_index("c")
    sid = lax.axis_index("s")
    wid = sid * NC + cid
    pltpu.sync_copy(ones_hbm, ones_v)
    pltpu.sync_copy(zeros_hbm, acc.at[pl.ds(sid * RPT, RPT)])
    plsc.subcore_barrier()
    wbase = wid * EPW
    semS = (semS0, semS1)

    def do_group(base, p, drain_first):
        if drain_first:
            for j in range(K):
                pltpu.make_async_copy(
                    ones_v, acc.at[didx_v.at[p, j]], semS[p]).wait()
        loads = [pltpu.async_copy(
            dst_hbm.at[pl.ds(base + j * CH, CH)], didx_v.at[p, j], semI)
            for j in range(K)]
        for d in loads:
            d.wait()
        for j in range(K):
            pltpu.async_copy(ones_v, acc.at[didx_v.at[p, j]], semS[p],
                             add=True)

    do_group(wbase, 0, False)
    do_group(wbase + GRP, 1, False)

    def body(i, carry):
        do_group(wbase + (2 * i + 2) * GRP, 0, True)
        do_group(wbase + (2 * i + 3) * GRP, 1, True)
        return carry

    lax.fori_loop(0, (NGRP - 2) // 2, body, 0)
    for p in (1, 0):
        for j in range(K):
            pltpu.make_async_copy(
                ones_v, acc.at[didx_v.at[p, j]], semS[p]).wait()
    plsc.subcore_barrier()
    pltpu.sync_copy(acc.at[pl.ds(sid * RPT, RPT)],
                    out_hbm.at[cid, pl.ds(sid * RPT, RPT)])


@functools.cache
def _sc_aggregate_kernel():
    mesh = plsc.VectorSubcoreMesh(core_axis_name="c", subcore_axis_name="s",
                                  num_cores=NC, num_subcores=NS)
    return pl.kernel(
        _sc_aggregate,
        out_type=jax.ShapeDtypeStruct((NC, NPAD, D), jnp.float32),
        mesh=mesh,
        scratch_types=[
            pltpu.VMEM((AK, ACH), jnp.int32),
            pltpu.VMEM((AK, ACH), jnp.int32),
            pltpu.VMEM((AK, ACH), jnp.int32),
            pltpu.VMEM((AK, ACH), jnp.int32),
            pltpu.VMEM((AK, ACH, D), jnp.float32),
            pltpu.VMEM((AK, ACH, D), jnp.float32),
            pltpu.VMEM_SHARED((NPAD, D), jnp.float32),
            pltpu.SemaphoreType.DMA,
            pltpu.SemaphoreType.DMA,
            pltpu.SemaphoreType.DMA,
            pltpu.SemaphoreType.DMA,
        ],
    )


def _sc_aggregate(z_hbm, src_hbm, dst_hbm, zeros_hbm, out_hbm,
                  sidx0, sidx1, didx0, didx1, rows0, rows1, acc,
                  semI, semG, semS0, semS1):
    cid = lax.axis_index("c")
    sid = lax.axis_index("s")
    wid = sid * NC + cid
    pltpu.sync_copy(zeros_hbm, acc.at[pl.ds(sid * RPT, RPT)])
    plsc.subcore_barrier()
    wrow = wid * EPW
    sidx = (sidx0, sidx1)
    didx = (didx0, didx1)
    rows = (rows0, rows1)
    semS = (semS0, semS1)

    def do_group(rb, p, drain_first):
        if drain_first:
            for j in range(AK):
                pltpu.make_async_copy(
                    rows[p].at[j], acc.at[didx[p].at[j]], semS[p]).wait()
        loads = [pltpu.async_copy(
            dst_hbm.at[pl.ds(rb + j * ACH, ACH)], didx[p].at[j], semI)
            for j in range(AK)]
        loads += [pltpu.async_copy(
            src_hbm.at[pl.ds(rb + j * ACH, ACH)], sidx[p].at[j], semI)
            for j in range(AK)]
        for d in loads:
            d.wait()
        gathers = [pltpu.async_copy(
            z_hbm.at[sidx[p].at[j]], rows[p].at[j], semG) for j in range(AK)]
        for d in gathers:
            d.wait()
        for j in range(AK):
            pltpu.async_copy(rows[p].at[j], acc.at[didx[p].at[j]],
                             semS[p], add=True)

    do_group(wrow, 0, False)
    do_group(wrow + AGRP, 1, False)

    def body(i, carry):
        do_group(wrow + (2 * i + 2) * AGRP, 0, True)
        do_group(wrow + (2 * i + 3) * AGRP, 1, True)
        return carry

    lax.fori_loop(0, (ANGRP - 2) // 2, body, 0)
    for p in (1, 0):
        for j in range(AK):
            pltpu.make_async_copy(
                rows[p].at[j], acc.at[didx[p].at[j]], semS[p]).wait()
    plsc.subcore_barrier()
    pltpu.sync_copy(acc.at[pl.ds(sid * RPT, RPT)],
                    out_hbm.at[cid, pl.ds(sid * RPT, RPT)])


def _dinv(deg_ref):
    deg = deg_ref[0] + deg_ref[1] + 1.0
    return lax.rsqrt(deg)


def _tc1_body(x_ref, win_ref, bin_ref, wg1_ref, deg_ref, z1_ref):
    x0 = jnp.dot(x_ref[...], win_ref[...],
                 preferred_element_type=jnp.float32) + bin_ref[...]
    y1 = jnp.dot(x0, wg1_ref[...], preferred_element_type=jnp.float32)
    z1_ref[...] = jnp.concatenate(
        [y1 * _dinv(deg_ref), jnp.zeros((8, H), jnp.float32)], axis=0)


def _bn(h, gamma, beta):
    m = jnp.mean(h, axis=0, keepdims=True)
    v = jnp.mean((h - m) ** 2, axis=0, keepdims=True)
    return (h - m) / jnp.sqrt(v + EPS) * gamma + beta


def _tc2_body(aggp_ref, z1_ref, deg_ref, bg1_ref, g1_ref, b1_ref, wg2_ref,
              z2_ref):
    dinv = _dinv(deg_ref)
    h = (aggp_ref[0] + aggp_ref[1] + z1_ref[0:N]) * dinv + bg1_ref[...]
    h = _bn(h, g1_ref[...], b1_ref[...])
    h = jnp.maximum(h, 0.0)
    y2 = jnp.dot(h, wg2_ref[...], preferred_element_type=jnp.float32)
    z2_ref[...] = jnp.concatenate(
        [y2 * dinv, jnp.zeros((8, H), jnp.float32)], axis=0)


def _tc3_body(aggp_ref, z2_ref, deg_ref, bg2_ref, g2_ref, b2_ref,
              wo1_ref, bo1_ref, wo2_ref, bo2_ref, out_ref):
    dinv = _dinv(deg_ref)
    h = (aggp_ref[0] + aggp_ref[1] + z2_ref[0:N]) * dinv + bg2_ref[...]
    h = _bn(h, g2_ref[...], b2_ref[...])
    o = jnp.maximum(
        jnp.dot(h, wo1_ref[...], preferred_element_type=jnp.float32)
        + bo1_ref[...], 0.0)
    out_ref[...] = (jnp.dot(o, wo2_ref[...],
                            preferred_element_type=jnp.float32)
                    + bo2_ref[...])


def kernel(X, edge_index, W_in, b_in, W_g1, b_g1, gamma1, beta1,
           W_g2, b_g2, gamma2, beta2, W_o1, b_o1, W_o2, b_o2):
    ppw = EPW - E // NW
    src = jnp.concatenate(
        [edge_index[0].reshape(NW, E // NW),
         jnp.full((NW, ppw), N, jnp.int32)], axis=1).reshape(-1)
    spread = (jnp.arange(NW * ppw, dtype=jnp.int32) * 1031) % NPAD
    dst = jnp.concatenate(
        [edge_index[1].reshape(NW, E // NW), spread.reshape(NW, ppw)],
        axis=1).reshape(-1)
    pad_rows = jnp.broadcast_to(N + jnp.arange(ppw, dtype=jnp.int32),
                                (NW, ppw))
    dst_deg = jnp.concatenate(
        [edge_index[1].reshape(NW, E // NW), pad_rows], axis=1).reshape(-1)
    zerosD = jnp.zeros((RPT, D), jnp.float32)
    onesD = jnp.ones((CH, D), jnp.float32)

    degp = _sc_degree_kernel()(dst_deg, onesD, zerosD)
    deg2 = degp[:, :N, 0:1]

    z1 = pl.pallas_call(
        _tc1_body,
        out_shape=jax.ShapeDtypeStruct((N + 8, H), jnp.float32),
    )(X, W_in, b_in.reshape(1, H), W_g1, deg2)

    agg1p = _sc_aggregate_kernel()(z1, src, dst, zerosD)

    z2 = pl.pallas_call(
        _tc2_body,
        out_shape=jax.ShapeDtypeStruct((N + 8, H), jnp.float32),
    )(agg1p[:, :N, :], z1, deg2, b_g1.reshape(1, H), gamma1.reshape(1, H),
      beta1.reshape(1, H), W_g2)

    agg2p = _sc_aggregate_kernel()(z2, src, dst, zerosD)

    out = pl.pallas_call(
        _tc3_body,
        out_shape=jax.ShapeDtypeStruct((N, OUT), jnp.float32),
    )(agg2p[:, :N, :], z2, deg2, b_g2.reshape(1, H), gamma2.reshape(1, H),
      beta2.reshape(1, H), W_o1, b_o1.reshape(1, END), W_o2,
      b_o2.reshape(1, OUT))
    return out


# speedup vs baseline: 2.1826x; 2.1826x over previous
import functools

import jax
import jax.numpy as jnp
from jax import lax
from jax.experimental import pallas as pl
from jax.experimental.pallas import tpu as pltpu
from jax.experimental.pallas import tpu_sc as plsc

N = 10000
E = 320000
D = 128
H = 128
END = 64
OUT = 1
EPS = 1e-5

NC = 2
NS = 16
NW = NC * NS
CH = 128
EPAD = 327680
EPW = EPAD // NW
NPAD = 10240
RPT = NPAD // NS
DUMP = NPAD - 1

K = 5
GRP = K * CH
NGRP = EPW // GRP
AK = 5
ACH = 32
AGRP = AK * ACH
ANGRP = EPW // AGRP
WROWS = EPW // ACH


@functools.cache
def _sc_degree_kernel():
    mesh = plsc.VectorSubcoreMesh(core_axis_name="c", subcore_axis_name="s",
                                  num_cores=NC, num_subcores=NS)
    return pl.kernel(
        _sc_degree,
        out_type=jax.ShapeDtypeStruct((NC, NPAD, D), jnp.float32),
        mesh=mesh,
        scratch_types=[
            pltpu.VMEM((2, K, CH), jnp.int32),
            pltpu.VMEM((CH, D), jnp.float32),
            pltpu.VMEM_SHARED((NPAD, D), jnp.float32),
            pltpu.SemaphoreType.DMA,
            pltpu.SemaphoreType.DMA,
            pltpu.SemaphoreType.DMA,
        ],
    )


def _sc_degree(dst_hbm, ones_hbm, zeros_hbm, out_hbm, didx_v, ones_v, acc,
               semI, semS0, semS1):
    cid = lax.axis_index("c")
    sid = lax.axis_index("s")
    wid = sid * NC + cid
    pltpu.sync_copy(ones_hbm, ones_v)
    pltpu.sync_copy(zeros_hbm, acc.at[pl.ds(sid * RPT, RPT)])
    plsc.subcore_barrier()
    wbase = wid * EPW
    semS = (semS0, semS1)

    def do_group(base, p, drain_first):
        if drain_first:
            for j in range(K):
                pltpu.make_async_copy(
                    ones_v, acc.at[didx_v.at[p, j]], semS[p]).wait()
        loads = [pltpu.async_copy(
            dst_hbm.at[pl.ds(base + j * CH, CH)], didx_v.at[p, j], semI)
            for j in range(K)]
        for d in loads:
            d.wait()
        for j in range(K):
            pltpu.async_copy(ones_v, acc.at[didx_v.at[p, j]], semS[p],
                             add=True)

    do_group(wbase, 0, False)
    do_group(wbase + GRP, 1, False)

    def body(i, carry):
        do_group(wbase + (2 * i + 2) * GRP, 0, True)
        do_group(wbase + (2 * i + 3) * GRP, 1, True)
        return carry

    lax.fori_loop(0, (NGRP - 2) // 2, body, 0)
    for p in (1, 0):
        for j in range(K):
            pltpu.make_async_copy(
                ones_v, acc.at[didx_v.at[p, j]], semS[p]).wait()
    plsc.subcore_barrier()
    pltpu.sync_copy(acc.at[pl.ds(sid * RPT, RPT)],
                    out_hbm.at[cid, pl.ds(sid * RPT, RPT)])


@functools.cache
def _sc_aggregate_kernel():
    mesh = plsc.VectorSubcoreMesh(core_axis_name="c", subcore_axis_name="s",
                                  num_cores=NC, num_subcores=NS)
    return pl.kernel(
        _sc_aggregate,
        out_type=jax.ShapeDtypeStruct((NC, NPAD, D), jnp.float32),
        mesh=mesh,
        scratch_types=[
            pltpu.VMEM((AK, ACH), jnp.int32),
            pltpu.VMEM((AK, ACH), jnp.int32),
            pltpu.VMEM((AK, ACH), jnp.int32),
            pltpu.VMEM((AK, ACH), jnp.int32),
            pltpu.VMEM((AK, ACH, D), jnp.float32),
            pltpu.VMEM((AK, ACH, D), jnp.float32),
            pltpu.VMEM_SHARED((NPAD, D), jnp.float32),
            pltpu.SemaphoreType.DMA,
            pltpu.SemaphoreType.DMA,
            pltpu.SemaphoreType.DMA,
            pltpu.SemaphoreType.DMA,
        ],
    )


def _sc_aggregate(z_hbm, src_hbm, dst_hbm, zeros_hbm, out_hbm,
                  sidx0, sidx1, didx0, didx1, rows0, rows1, acc,
                  semI, semG, semS0, semS1):
    cid = lax.axis_index("c")
    sid = lax.axis_index("s")
    wid = sid * NC + cid
    pltpu.sync_copy(zeros_hbm, acc.at[pl.ds(sid * RPT, RPT)])
    plsc.subcore_barrier()
    wrow = wid * EPW
    sidx = (sidx0, sidx1)
    didx = (didx0, didx1)
    rows = (rows0, rows1)
    semS = (semS0, semS1)

    def do_group(rb, p, drain_first):
        if drain_first:
            for j in range(AK):
                pltpu.make_async_copy(
                    rows[p].at[j], acc.at[didx[p].at[j]], semS[p]).wait()
        loads = [pltpu.async_copy(
            dst_hbm.at[pl.ds(rb + j * ACH, ACH)], didx[p].at[j], semI)
            for j in range(AK)]
        loads += [pltpu.async_copy(
            src_hbm.at[pl.ds(rb + j * ACH, ACH)], sidx[p].at[j], semI)
            for j in range(AK)]
        for d in loads:
            d.wait()
        gathers = [pltpu.async_copy(
            z_hbm.at[sidx[p].at[j]], rows[p].at[j], semG) for j in range(AK)]
        for d in gathers:
            d.wait()
        for j in range(AK):
            pltpu.async_copy(rows[p].at[j], acc.at[didx[p].at[j]],
                             semS[p], add=True)

    do_group(wrow, 0, False)
    do_group(wrow + AGRP, 1, False)

    def body(i, carry):
        do_group(wrow + (2 * i + 2) * AGRP, 0, True)
        do_group(wrow + (2 * i + 3) * AGRP, 1, True)
        return carry

    lax.fori_loop(0, (ANGRP - 2) // 2, body, 0)
    for p in (1, 0):
        for j in range(AK):
            pltpu.make_async_copy(
                rows[p].at[j], acc.at[didx[p].at[j]], semS[p]).wait()
    plsc.subcore_barrier()
    pltpu.sync_copy(acc.at[pl.ds(sid * RPT, RPT)],
                    out_hbm.at[cid, pl.ds(sid * RPT, RPT)])


def _dinv(deg_ref):
    deg = deg_ref[0] + deg_ref[1] + 1.0
    return lax.rsqrt(deg)


def _tc1_body(x_ref, win_ref, bin_ref, wg1_ref, deg_ref, z1_ref):
    x0 = jnp.dot(x_ref[...], win_ref[...],
                 preferred_element_type=jnp.float32) + bin_ref[...]
    y1 = jnp.dot(x0, wg1_ref[...], preferred_element_type=jnp.float32)
    z1_ref[...] = y1 * _dinv(deg_ref)


def _bn(h, gamma, beta):
    m = jnp.mean(h, axis=0, keepdims=True)
    v = jnp.mean((h - m) ** 2, axis=0, keepdims=True)
    return (h - m) / jnp.sqrt(v + EPS) * gamma + beta


def _tc2_body(aggp_ref, z1_ref, deg_ref, bg1_ref, g1_ref, b1_ref, wg2_ref,
              z2_ref):
    dinv = _dinv(deg_ref)
    h = (aggp_ref[0] + aggp_ref[1] + z1_ref[...]) * dinv + bg1_ref[...]
    h = _bn(h, g1_ref[...], b1_ref[...])
    h = jnp.maximum(h, 0.0)
    y2 = jnp.dot(h, wg2_ref[...], preferred_element_type=jnp.float32)
    z2_ref[...] = y2 * dinv


def _tc3_body(aggp_ref, z2_ref, deg_ref, bg2_ref, g2_ref, b2_ref,
              wo1_ref, bo1_ref, wo2_ref, bo2_ref, out_ref):
    dinv = _dinv(deg_ref)
    h = (aggp_ref[0] + aggp_ref[1] + z2_ref[...]) * dinv + bg2_ref[...]
    h = _bn(h, g2_ref[...], b2_ref[...])
    o = jnp.maximum(
        jnp.dot(h, wo1_ref[...], preferred_element_type=jnp.float32)
        + bo1_ref[...], 0.0)
    out_ref[...] = (jnp.dot(o, wo2_ref[...],
                            preferred_element_type=jnp.float32)
                    + bo2_ref[...])


def kernel(X, edge_index, W_in, b_in, W_g1, b_g1, gamma1, beta1,
           W_g2, b_g2, gamma2, beta2, W_o1, b_o1, W_o2, b_o2):
    ppw = EPW - E // NW
    pad_src = (jnp.arange(NW * ppw, dtype=jnp.int32) * 997) % N
    src = jnp.concatenate(
        [edge_index[0].reshape(NW, E // NW), pad_src.reshape(NW, ppw)],
        axis=1).reshape(-1)
    pad_rows = jnp.broadcast_to(N + jnp.arange(ppw, dtype=jnp.int32),
                                (NW, ppw))
    dst = jnp.concatenate(
        [edge_index[1].reshape(NW, E // NW), pad_rows], axis=1).reshape(-1)

    zerosD = jnp.zeros((RPT, D), jnp.float32)
    onesD = jnp.ones((CH, D), jnp.float32)

    degp = _sc_degree_kernel()(dst, onesD, zerosD)
    deg2 = degp[:, :N, 0:1]

    z1 = pl.pallas_call(
        _tc1_body,
        out_shape=jax.ShapeDtypeStruct((N, H), jnp.float32),
    )(X, W_in, b_in.reshape(1, H), W_g1, deg2)

    agg1p = _sc_aggregate_kernel()(z1, src, dst, zerosD)

    z2 = pl.pallas_call(
        _tc2_body,
        out_shape=jax.ShapeDtypeStruct((N, H), jnp.float32),
    )(agg1p[:, :N, :], z1, deg2, b_g1.reshape(1, H), gamma1.reshape(1, H),
      beta1.reshape(1, H), W_g2)

    agg2p = _sc_aggregate_kernel()(z2, src, dst, zerosD)

    out = pl.pallas_call(
        _tc3_body,
        out_shape=jax.ShapeDtypeStruct((N, OUT), jnp.float32),
    )(agg2p[:, :N, :], z2, deg2, b_g2.reshape(1, H), gamma2.reshape(1, H),
      beta2.reshape(1, H), W_o1, b_o1.reshape(1, END), W_o2,
      b_o2.reshape(1, OUT))
    return out
